# Initial kernel scaffold; baseline (speedup 1.0000x reference)
import jax
import jax.numpy as jnp
from jax.experimental import pallas as pl


def kernel(rel_seq, pos_seq, rel_table, pos_table):
    raise NotImplementedError("write your pallas kernel here")


# speedup vs baseline: 1.6888x; 1.6888x over previous
import jax
import jax.numpy as jnp
from jax import lax
from jax.experimental import pallas as pl
from jax.experimental.pallas import tpu as pltpu
from jax.experimental.pallas import tpu_sc as plsc

EMBED = 64
LANES = 16
NC, NS = 2, 16
NW = NC * NS
K = 128

_mesh = plsc.VectorSubcoreMesh(
    core_axis_name="c", subcore_axis_name="s", num_cores=NC, num_subcores=NS
)


def _body(rel_idx, pos_idx, rel_tab, pos_tab, out, idx_v, pidx_v,
          rel_rows, pos_rows, sem):
    n = rel_idx.shape[0]
    n_per_w = n // NW
    iters = n_per_w // K
    wid = lax.axis_index("s") * NC + lax.axis_index("c")
    base0 = wid * n_per_w

    def step(i, carry):
        base = base0 + i * K
        pltpu.sync_copy(rel_idx.at[pl.ds(base, K)], idx_v)
        pltpu.sync_copy(pos_idx.at[pl.ds(base, K)], pidx_v)
        c1 = pltpu.async_copy(rel_tab.at[idx_v], rel_rows, sem)
        c2 = pltpu.async_copy(pos_tab.at[pidx_v], pos_rows, sem)
        c1.wait()
        c2.wait()

        def row(r, rcarry):
            for cc in range(EMBED // LANES):
                sl = pl.ds(cc * LANES, LANES)
                rel_rows[r, sl] = rel_rows[r, sl] + pos_rows[r, sl]
            return rcarry

        lax.fori_loop(0, K, row, 0, unroll=2)
        pltpu.sync_copy(rel_rows, out.at[pl.ds(base, K)])
        return carry

    lax.fori_loop(0, iters, step, 0)


def kernel(rel_seq, pos_seq, rel_table, pos_table):
    b, s = rel_seq.shape
    n = b * s
    run = pl.kernel(
        _body,
        out_type=jax.ShapeDtypeStruct((n, EMBED), jnp.float32),
        mesh=_mesh,
        scratch_types=[
            pltpu.VMEM((K,), jnp.int32),
            pltpu.VMEM((K,), jnp.int32),
            pltpu.VMEM((K, EMBED), jnp.float32),
            pltpu.VMEM((K, EMBED), jnp.float32),
            pltpu.SemaphoreType.DMA,
        ],
        compiler_params=pltpu.CompilerParams(use_tc_tiling_on_sc=False),
    )
    out = run(rel_seq.reshape(n), pos_seq.reshape(n), rel_table, pos_table)
    return out.reshape(b, s, EMBED)


# speedup vs baseline: 1.9905x; 1.1787x over previous
import jax
import jax.numpy as jnp
from jax import lax
from jax.experimental import pallas as pl
from jax.experimental.pallas import tpu as pltpu
from jax.experimental.pallas import tpu_sc as plsc

EMBED = 64
LANES = 16
POS_ROWS = 512
NC, NS = 2, 16
NW = NC * NS
K = 128

_mesh = plsc.VectorSubcoreMesh(
    core_axis_name="c", subcore_axis_name="s", num_cores=NC, num_subcores=NS
)


def _body(rel_idx, pos_idx, rel_tab, pos_tab, out,
          pos_v, idx2, pidx2, rows2, sem_idx, sem_g, sem_out):
    n = rel_idx.shape[0]
    n_per_w = n // NW
    nchunk = n_per_w // K
    wid = lax.axis_index("s") * NC + lax.axis_index("c")
    base0 = wid * n_per_w

    def fire_idx(i, slot):
        base = base0 + i * K
        pltpu.async_copy(rel_idx.at[pl.ds(base, K)], idx2.at[slot], sem_idx.at[slot])
        pltpu.async_copy(pos_idx.at[pl.ds(base, K)], pidx2.at[slot], sem_idx.at[slot])

    def wait_idx(slot):
        pltpu.make_async_copy(
            rel_idx.at[pl.ds(0, K)], idx2.at[slot], sem_idx.at[slot]).wait()
        pltpu.make_async_copy(
            pos_idx.at[pl.ds(0, K)], pidx2.at[slot], sem_idx.at[slot]).wait()

    def fire_gather(slot):
        pltpu.async_copy(rel_tab.at[idx2.at[slot]], rows2.at[slot], sem_g)

    def wait_gather(slot):
        pltpu.make_async_copy(rel_tab.at[idx2.at[slot]], rows2.at[slot], sem_g).wait()

    def fire_out(i, slot):
        base = base0 + i * K
        pltpu.async_copy(rows2.at[slot], out.at[pl.ds(base, K)], sem_out)

    def wait_out(i, slot):
        base = base0 + i * K
        pltpu.make_async_copy(rows2.at[slot], out.at[pl.ds(base, K)], sem_out).wait()

    pltpu.sync_copy(pos_tab, pos_v)

    fire_idx(0, 0)
    fire_idx(1, 1)
    wait_idx(0)
    fire_gather(0)

    iota = lax.iota(jnp.int32, LANES)

    def step(i, carry):
        s = lax.rem(i, 2)
        ns = 1 - s
        wait_gather(s)

        @pl.when(i >= 1)
        def _():
            wait_out(i - 1, ns)

        @pl.when(i + 1 < nchunk)
        def _():
            wait_idx(ns)
            fire_gather(ns)

        rr = rows2.at[s]
        pp = pidx2.at[s]

        def row(r, rcarry):
            pb = plsc.load_gather(pp, [jnp.full((LANES,), r, jnp.int32)])
            for c in range(EMBED // LANES):
                v = plsc.load_gather(pos_v, [pb, iota + (c * LANES)])
                plsc.addupdate(rr.at[r, pl.ds(c * LANES, LANES)], v)
            return rcarry

        lax.fori_loop(0, K, row, 0, unroll=2)
        fire_out(i, s)

        @pl.when(i + 2 < nchunk)
        def _():
            fire_idx(i + 2, s)

        return carry

    lax.fori_loop(0, nchunk, step, 0)
    wait_out(nchunk - 1, lax.rem(nchunk - 1, 2))


def kernel(rel_seq, pos_seq, rel_table, pos_table):
    b, s = rel_seq.shape
    n = b * s
    run = pl.kernel(
        _body,
        out_type=jax.ShapeDtypeStruct((n, EMBED), jnp.float32),
        mesh=_mesh,
        scratch_types=[
            pltpu.VMEM((POS_ROWS, EMBED), jnp.float32),
            pltpu.VMEM((2, K), jnp.int32),
            pltpu.VMEM((2, K), jnp.int32),
            pltpu.VMEM((2, K, EMBED), jnp.float32),
            pltpu.SemaphoreType.DMA((2,)),
            pltpu.SemaphoreType.DMA,
            pltpu.SemaphoreType.DMA,
        ],
        compiler_params=pltpu.CompilerParams(
            use_tc_tiling_on_sc=False, needs_layout_passes=False
        ),
    )
    out = run(rel_seq.reshape(n), pos_seq.reshape(n), rel_table, pos_table)
    return out.reshape(b, s, EMBED)


# speedup vs baseline: 2.0266x; 1.0181x over previous
import jax
import jax.numpy as jnp
from jax import lax
from jax.experimental import pallas as pl
from jax.experimental.pallas import tpu as pltpu
from jax.experimental.pallas import tpu_sc as plsc

EMBED = 64
LANES = 16
POS_ROWS = 512
NC, NS = 2, 16
NW = NC * NS
SEQ = 200
G1 = 128
G2 = SEQ - G1

_mesh = plsc.VectorSubcoreMesh(
    core_axis_name="c", subcore_axis_name="s", num_cores=NC, num_subcores=NS
)


def _body(rel_idx, pos_idx, rel_tab, pos_tab, out,
          pos_v, idx2, pidx2, rows2, sem_idx, sem_g, sem_out):
    n = rel_idx.shape[0]
    nb = out.shape[0]
    b_per_w = nb // NW
    wid = lax.axis_index("s") * NC + lax.axis_index("c")
    b0 = wid * b_per_w

    def fire_idx(i, slot):
        base = (b0 + i) * SEQ
        pltpu.async_copy(rel_idx.at[pl.ds(base, SEQ)],
                         idx2.at[slot, pl.ds(0, SEQ)], sem_idx.at[slot])
        pltpu.async_copy(pos_idx.at[pl.ds(base, SEQ)],
                         pidx2.at[slot, pl.ds(0, SEQ)], sem_idx.at[slot])

    def wait_idx(slot):
        pltpu.make_async_copy(rel_idx.at[pl.ds(0, SEQ)],
                              idx2.at[slot, pl.ds(0, SEQ)], sem_idx.at[slot]).wait()
        pltpu.make_async_copy(pos_idx.at[pl.ds(0, SEQ)],
                              pidx2.at[slot, pl.ds(0, SEQ)], sem_idx.at[slot]).wait()

    def fire_gather(slot):
        pltpu.async_copy(rel_tab.at[idx2.at[slot, pl.ds(0, G1)]],
                         rows2.at[slot, pl.ds(0, G1)], sem_g)
        pltpu.async_copy(rel_tab.at[idx2.at[slot, pl.ds(G1, G2)]],
                         rows2.at[slot, pl.ds(G1, G2)], sem_g)

    def wait_gather(slot):
        pltpu.make_async_copy(rel_tab.at[idx2.at[slot, pl.ds(0, G1)]],
                              rows2.at[slot, pl.ds(0, G1)], sem_g).wait()
        pltpu.make_async_copy(rel_tab.at[idx2.at[slot, pl.ds(G1, G2)]],
                              rows2.at[slot, pl.ds(G1, G2)], sem_g).wait()

    def fire_out(i, slot):
        pltpu.async_copy(rows2.at[slot], out.at[b0 + i], sem_out)

    def wait_out(i, slot):
        pltpu.make_async_copy(rows2.at[slot], out.at[b0 + i], sem_out).wait()

    pltpu.sync_copy(pos_tab, pos_v)

    fire_idx(0, 0)
    fire_idx(1, 1)
    wait_idx(0)
    fire_gather(0)

    iota = lax.iota(jnp.int32, LANES)

    def step(i, carry):
        s = lax.rem(i, 2)
        ns = 1 - s
        wait_gather(s)

        @pl.when(i >= 1)
        def _():
            wait_out(i - 1, ns)

        @pl.when(i + 1 < b_per_w)
        def _():
            wait_idx(ns)
            fire_gather(ns)

        rr = rows2.at[s]
        pp = pidx2.at[s]

        def row(r, rcarry):
            pb = plsc.load_gather(pp, [jnp.full((LANES,), r, jnp.int32)])
            for c in range(EMBED // LANES):
                v = plsc.load_gather(pos_v, [pb, iota + (c * LANES)])
                plsc.addupdate(rr.at[r, pl.ds(c * LANES, LANES)], v)
            return rcarry

        lax.fori_loop(0, SEQ, row, 0, unroll=2)
        fire_out(i, s)

        @pl.when(i + 2 < b_per_w)
        def _():
            fire_idx(i + 2, s)

        return carry

    lax.fori_loop(0, b_per_w, step, 0)
    wait_out(b_per_w - 1, lax.rem(b_per_w - 1, 2))


def kernel(rel_seq, pos_seq, rel_table, pos_table):
    b, s = rel_seq.shape
    n = b * s
    run = pl.kernel(
        _body,
        out_type=jax.ShapeDtypeStruct((b, s, EMBED), jnp.float32),
        mesh=_mesh,
        scratch_types=[
            pltpu.VMEM((POS_ROWS, EMBED), jnp.float32),
            pltpu.VMEM((2, 256), jnp.int32),
            pltpu.VMEM((2, 256), jnp.int32),
            pltpu.VMEM((2, SEQ, EMBED), jnp.float32),
            pltpu.SemaphoreType.DMA((2,)),
            pltpu.SemaphoreType.DMA,
            pltpu.SemaphoreType.DMA,
        ],
        compiler_params=pltpu.CompilerParams(
            use_tc_tiling_on_sc=False, needs_layout_passes=False
        ),
    )
    return run(rel_seq.reshape(n), pos_seq.reshape(n), rel_table, pos_table)


# speedup vs baseline: 2.1176x; 1.0449x over previous
import jax
import jax.numpy as jnp
from jax import lax
from jax.experimental import pallas as pl
from jax.experimental.pallas import tpu as pltpu
from jax.experimental.pallas import tpu_sc as plsc

EMBED = 64
LANES = 16
POS_ROWS = 512
NC, NS = 2, 16
NW = NC * NS
SEQ = 200
G1 = 128
G2 = SEQ - G1
NBUF = 4

_mesh = plsc.VectorSubcoreMesh(
    core_axis_name="c", subcore_axis_name="s", num_cores=NC, num_subcores=NS
)


def _body(rel_idx, pos_idx, rel_tab, pos_tab, out,
          pos_v, idx2, pidx2, rows2, sem_idx, sem_g, sem_out):
    n = rel_idx.shape[0]
    nb = out.shape[0]
    b_per_w = nb // NW
    wid = lax.axis_index("s") * NC + lax.axis_index("c")
    b0 = wid * b_per_w

    def fire_idx(i, slot):
        base = (b0 + i) * SEQ
        pltpu.async_copy(rel_idx.at[pl.ds(base, SEQ)],
                         idx2.at[slot, pl.ds(0, SEQ)], sem_idx.at[slot])
        pltpu.async_copy(pos_idx.at[pl.ds(base, SEQ)],
                         pidx2.at[slot, pl.ds(0, SEQ)], sem_idx.at[slot])

    def wait_idx(slot):
        pltpu.make_async_copy(rel_idx.at[pl.ds(0, SEQ)],
                              idx2.at[slot, pl.ds(0, SEQ)], sem_idx.at[slot]).wait()
        pltpu.make_async_copy(pos_idx.at[pl.ds(0, SEQ)],
                              pidx2.at[slot, pl.ds(0, SEQ)], sem_idx.at[slot]).wait()

    def fire_gather(slot):
        pltpu.async_copy(rel_tab.at[idx2.at[slot, pl.ds(0, G1)]],
                         rows2.at[slot, pl.ds(0, G1)], sem_g.at[slot])
        pltpu.async_copy(rel_tab.at[idx2.at[slot, pl.ds(G1, G2)]],
                         rows2.at[slot, pl.ds(G1, G2)], sem_g.at[slot])

    def wait_gather(slot):
        pltpu.make_async_copy(rel_tab.at[idx2.at[slot, pl.ds(0, G1)]],
                              rows2.at[slot, pl.ds(0, G1)], sem_g.at[slot]).wait()
        pltpu.make_async_copy(rel_tab.at[idx2.at[slot, pl.ds(G1, G2)]],
                              rows2.at[slot, pl.ds(G1, G2)], sem_g.at[slot]).wait()

    def fire_out(i, slot):
        pltpu.async_copy(rows2.at[slot], out.at[b0 + i], sem_out.at[slot])

    def wait_out(i, slot):
        pltpu.make_async_copy(rows2.at[slot], out.at[b0 + i], sem_out.at[slot]).wait()

    pltpu.sync_copy(pos_tab, pos_v)

    fire_idx(0, 0)
    fire_idx(1, 1)
    fire_idx(2, 2)
    wait_idx(0)
    fire_gather(0)
    wait_idx(1)
    fire_gather(1)

    iota = lax.iota(jnp.int32, LANES)

    def step(i, carry):
        s = lax.rem(i, NBUF)
        wait_gather(s)

        @pl.when(i + 3 < b_per_w)
        def _():
            fire_idx(i + 3, lax.rem(i + 3, NBUF))

        @pl.when(i + 2 < b_per_w)
        def _():
            s2 = lax.rem(i + 2, NBUF)

            @pl.when(i >= 2)
            def _():
                wait_out(i - 2, s2)

            wait_idx(s2)
            fire_gather(s2)

        rr = rows2.at[s]
        pp = pidx2.at[s]

        def row(r, rcarry):
            pb = plsc.load_gather(pp, [jnp.full((LANES,), r, jnp.int32)])
            for c in range(EMBED // LANES):
                v = plsc.load_gather(pos_v, [pb, iota + (c * LANES)])
                plsc.addupdate(rr.at[r, pl.ds(c * LANES, LANES)], v)
            return rcarry

        lax.fori_loop(0, SEQ, row, 0, unroll=2)
        fire_out(i, s)
        return carry

    lax.fori_loop(0, b_per_w, step, 0)
    for j in range(NBUF):
        i = b_per_w - NBUF + j
        wait_out(i, lax.rem(i, NBUF))


def kernel(rel_seq, pos_seq, rel_table, pos_table):
    b, s = rel_seq.shape
    n = b * s
    run = pl.kernel(
        _body,
        out_type=jax.ShapeDtypeStruct((b, s, EMBED), jnp.float32),
        mesh=_mesh,
        scratch_types=[
            pltpu.VMEM((POS_ROWS, EMBED), jnp.float32),
            pltpu.VMEM((NBUF, 256), jnp.int32),
            pltpu.VMEM((NBUF, 256), jnp.int32),
            pltpu.VMEM((NBUF, SEQ, EMBED), jnp.float32),
            pltpu.SemaphoreType.DMA((NBUF,)),
            pltpu.SemaphoreType.DMA((NBUF,)),
            pltpu.SemaphoreType.DMA((NBUF,)),
        ],
        compiler_params=pltpu.CompilerParams(
            use_tc_tiling_on_sc=False, needs_layout_passes=False
        ),
    )
    return run(rel_seq.reshape(n), pos_seq.reshape(n), rel_table, pos_table)
